# baseline (speedup 1.0000x reference)
import functools

import jax
import jax.numpy as jnp
from jax import lax
from jax.experimental import pallas as pl
from jax.experimental.pallas import tpu as pltpu
from jax.experimental.pallas import tpu_sc as plsc

N_PAD = 10240
EDGES = 320000
NC, NS = 2, 16
PER = EDGES // NS
VECS = PER // 16
UNROLL = 10

_mesh = plsc.VectorSubcoreMesh(
    core_axis_name="c", subcore_axis_name="s", num_cores=NC, num_subcores=NS
)


@functools.partial(
    pl.kernel,
    out_type=jax.ShapeDtypeStruct((NC * NS * N_PAD,), jnp.int32),
    mesh=_mesh,
    scratch_types=[
        pltpu.VMEM((PER,), jnp.int32),
        pltpu.VMEM((N_PAD,), jnp.int32),
        pltpu.SemaphoreType.DMA,
    ],
    compiler_params=pltpu.CompilerParams(
        use_tc_tiling_on_sc=False, needs_layout_passes=False
    ),
)
def _hist_sc(edge_hbm, part_hbm, idx_v, hist_v, sem):
    c = lax.axis_index("c")
    s = lax.axis_index("s")
    idx_cp = pltpu.async_copy(
        edge_hbm.at[c, pl.ds(s * PER, PER)], idx_v, sem
    )

    zeros16 = jnp.zeros((16,), jnp.int32)

    def zero_body(i, carry):
        base = pl.multiple_of(i * 128, 128)
        for k in range(8):
            hist_v[pl.ds(base + k * 16, 16)] = zeros16
        return carry

    lax.fori_loop(0, N_PAD // 128, zero_body, 0)
    idx_cp.wait()

    ones16 = jnp.full((16,), 1, jnp.int32)

    def acc_body(i, carry):
        base = pl.multiple_of(i * (16 * UNROLL), 16)
        for k in range(UNROLL):
            idx16 = idx_v[pl.ds(base + k * 16, 16)]
            plsc.addupdate_scatter(hist_v, [idx16], ones16)
        return carry

    lax.fori_loop(0, VECS // UNROLL, acc_body, 0)

    off = pl.multiple_of((c * NS + s) * N_PAD, 8)
    pltpu.sync_copy(hist_v, part_hbm.at[pl.ds(off, N_PAD)])


def _red_body(part_ref, out_ref):
    s = pl.program_id(1)

    @pl.when(s == 0)
    def _():
        out_ref[...] = part_ref[...]

    @pl.when(s != 0)
    def _():
        out_ref[...] += part_ref[...]


_reduce_tc = pl.pallas_call(
    _red_body,
    grid=(NC, NS),
    in_specs=[pl.BlockSpec((N_PAD,), lambda c, s: (c * NS + s,))],
    out_specs=pl.BlockSpec((N_PAD,), lambda c, s: (c,)),
    out_shape=jax.ShapeDtypeStruct((NC * N_PAD,), jnp.int32),
)


def kernel(x, edge_index):
    ei = edge_index.astype(jnp.int32)
    parts = _hist_sc(ei)
    deg = _reduce_tc(parts)
    out_dtype = jax.dtypes.canonicalize_dtype(jnp.int64)
    out_degree = deg[:10000].astype(out_dtype)
    in_degree = deg[N_PAD:N_PAD + 10000].astype(out_dtype)
    return x, in_degree, out_degree


# speedup vs baseline: 1.4558x; 1.4558x over previous
import functools

import jax
import jax.numpy as jnp
from jax import lax
from jax.experimental import pallas as pl
from jax.experimental.pallas import tpu as pltpu
from jax.experimental.pallas import tpu_sc as plsc

N_NODES_PAD = 10240
EDGES = 320000
NC, NS = 2, 16
PER_SUBCORE = EDGES // NS
NSTREAM = 5
STREAM = PER_SUBCORE // NSTREAM
SLICE = N_NODES_PAD // NS

_mesh = plsc.VectorSubcoreMesh(
    core_axis_name="c", subcore_axis_name="s", num_cores=NC, num_subcores=NS
)


@functools.partial(
    pl.kernel,
    out_type=jax.ShapeDtypeStruct((NC, N_NODES_PAD), jnp.int32),
    mesh=_mesh,
    scratch_types=[
        [pltpu.VMEM((STREAM,), jnp.int32) for _ in range(NSTREAM)],
        pltpu.VMEM((STREAM,), jnp.int32),
        pltpu.VMEM((SLICE,), jnp.int32),
        pltpu.VMEM_SHARED((N_NODES_PAD,), jnp.int32),
        pltpu.SemaphoreType.DMA,
        pltpu.SemaphoreType.DMA,
    ],
    compiler_params=pltpu.CompilerParams(use_tc_tiling_on_sc=False),
)
def _degree_sc(edge_hbm, deg_hbm, idx_v, ones_v, zero_v, hist_s, sem, sem2):
    c = lax.axis_index("c")
    s = lax.axis_index("s")

    idx_cps = [
        pltpu.async_copy(
            edge_hbm.at[c, pl.ds(s * PER_SUBCORE + k * STREAM, STREAM)],
            idx_v[k],
            sem,
        )
        for k in range(NSTREAM)
    ]

    def fill_ones(i, carry):
        ones_v[pl.ds(pl.multiple_of(i * 16, 16), 16)] = jnp.full((16,), 1, jnp.int32)
        return carry

    lax.fori_loop(0, STREAM // 16, fill_ones, 0)
    for k in range(SLICE // 16):
        zero_v[pl.ds(k * 16, 16)] = jnp.zeros((16,), jnp.int32)

    pltpu.sync_copy(zero_v, hist_s.at[pl.ds(s * SLICE, SLICE)])
    for cp in idx_cps:
        cp.wait()
    plsc.subcore_barrier()

    adds = [
        pltpu.async_copy(ones_v, hist_s.at[idx_v[k]], sem2, add=True)
        for k in range(NSTREAM)
    ]
    for cp in adds:
        cp.wait()
    plsc.subcore_barrier()

    pltpu.sync_copy(hist_s.at[pl.ds(s * SLICE, SLICE)],
                    deg_hbm.at[c, pl.ds(s * SLICE, SLICE)])


def kernel(x, edge_index):
    ei = edge_index.astype(jnp.int32)
    deg = _degree_sc(ei)
    out_dtype = jax.dtypes.canonicalize_dtype(jnp.int64)
    out_degree = deg[0, :10000].astype(out_dtype)
    in_degree = deg[1, :10000].astype(out_dtype)
    return x, in_degree, out_degree
